# baseline (speedup 1.0000x reference)
import functools

import jax
import jax.numpy as jnp
from jax import lax
from jax.experimental import pallas as pl
from jax.experimental.pallas import tpu as pltpu
from jax.experimental.pallas import tpu_sc as plsc

NSEG = 512
NROW = 320000
NCOL = 128
R_SC = 160000
R_TC = NROW - R_SC
NC = 2
NS = 16
CHALF = NCOL // NC
RPT = R_SC // NS
BLK = 125
NB = RPT // BLK
SEG_PT = NSEG // NS
CW = 16
BT = 2000
NBT = R_TC // BT


def _sc_body(x_hbm, ids_hbm, sums_hbm, counts_hbm,
             ids_v, xb0, xb1, xb2, xb3, ones_v, zb, zc, sbuf, tbuf, cbuf,
             ctbuf, obuf, tab_e, tab_o, cnt_e, cnt_o,
             sl0, sl1, sl2, sl3, ste, sto, sce, sco):
    c = lax.axis_index("c")
    s = lax.axis_index("s")
    col0 = c * CHALF
    row0 = s * RPT
    seg0 = s * SEG_PT

    xbs = (xb0, xb1, xb2, xb3)
    sls = (sl0, sl1, sl2, sl3)
    tabs = (tab_e, tab_o)
    cnts = (cnt_e, cnt_o)
    sts = (ste, sto)
    scs = (sce, sco)

    pltpu.sync_copy(ids_hbm.at[s], ids_v)

    one16 = jnp.ones((16,), jnp.float32)
    zero16 = jnp.zeros((16,), jnp.float32)
    for r in range(BLK):
        ones_v[r, :] = one16
    for r in range(SEG_PT):
        zc[r, :] = zero16
    for r in range(SEG_PT):
        for j in range(CHALF // 16):
            zb[r, pl.ds(j * 16, 16)] = zero16

    pltpu.sync_copy(zb, tab_e.at[pl.ds(seg0, SEG_PT), :])
    pltpu.sync_copy(zb, tab_o.at[pl.ds(seg0, SEG_PT), :])
    pltpu.sync_copy(zc, cnt_e.at[pl.ds(seg0, SEG_PT), :])
    pltpu.sync_copy(zc, cnt_o.at[pl.ds(seg0, SEG_PT), :])
    plsc.subcore_barrier()

    def start_load(b_idx, k):
        pltpu.async_copy(
            x_hbm.at[pl.ds(row0 + b_idx * BLK, BLK), pl.ds(col0, CHALF)],
            xbs[k], sls[k])

    def wait_load(k):
        pltpu.make_async_copy(
            x_hbm.at[pl.ds(0, BLK), pl.ds(0, CHALF)], xbs[k], sls[k]).wait()

    def start_scatter(b_idx, k):
        p = k % 2
        idx = ids_v.at[b_idx]
        pltpu.async_copy(ones_v, cnts[p].at[idx], scs[p], add=True)
        pltpu.async_copy(xbs[k], tabs[p].at[idx], sts[p], add=True)

    def wait_scatter(p):
        pltpu.make_async_copy(ones_v, cnts[p].at[ids_v.at[0]], scs[p]).wait()
        pltpu.make_async_copy(xbs[p], tabs[p].at[ids_v.at[0]], sts[p]).wait()

    start_load(0, 0)
    start_load(1, 1)
    start_load(2, 2)
    wait_load(0)
    start_scatter(0, 0)
    start_load(3, 3)
    wait_load(1)
    start_scatter(1, 1)
    wait_scatter(0)
    start_load(4, 0)
    wait_load(2)
    start_scatter(2, 2)
    wait_scatter(1)
    start_load(5, 1)
    wait_load(3)
    start_scatter(3, 3)

    def quad(i, carry):
        g = i * 4
        for k in range(4):
            wait_scatter(k % 2)
            start_load(g + k + 2, (k + 2) % 4)
            wait_load(k)
            start_scatter(g + k, k)
        return carry

    lax.fori_loop(1, NB // 4 - 1, quad, 0)

    for k in range(4):
        g = NB - 4 + k
        wait_scatter(k % 2)
        if k < 2:
            start_load(NB - 2 + k, k + 2)
        wait_load(k)
        start_scatter(g, k)
    wait_scatter(0)
    wait_scatter(1)

    plsc.subcore_barrier()

    pltpu.sync_copy(tab_e.at[pl.ds(seg0, SEG_PT), :], sbuf)
    pltpu.sync_copy(tab_o.at[pl.ds(seg0, SEG_PT), :], tbuf)
    pltpu.sync_copy(cnt_e.at[pl.ds(seg0, SEG_PT), :], cbuf)
    pltpu.sync_copy(cnt_o.at[pl.ds(seg0, SEG_PT), :], ctbuf)
    for r in range(SEG_PT):
        cbuf[r, :] += ctbuf[r, :]
        for j in range(CHALF // 16):
            sl = pl.ds(j * 16, 16)
            obuf[r, sl] = sbuf[r, sl] + tbuf[r, sl]
    pltpu.sync_copy(obuf, sums_hbm.at[pl.ds(seg0, SEG_PT), pl.ds(col0, CHALF)])
    pltpu.sync_copy(cbuf, counts_hbm.at[c, pl.ds(seg0, SEG_PT), :])


_sc_pool = functools.partial(
    pl.kernel,
    out_type=(jax.ShapeDtypeStruct((NSEG, NCOL), jnp.float32),
              jax.ShapeDtypeStruct((NC, NSEG, CW), jnp.float32)),
    mesh=plsc.VectorSubcoreMesh(core_axis_name="c", subcore_axis_name="s",
                                num_cores=NC, num_subcores=NS),
    compiler_params=pltpu.CompilerParams(use_tc_tiling_on_sc=False,
                                         needs_layout_passes=False),
    scratch_types=[
        pltpu.VMEM((NB, BLK), jnp.int32),
        pltpu.VMEM((BLK, CHALF), jnp.float32),
        pltpu.VMEM((BLK, CHALF), jnp.float32),
        pltpu.VMEM((BLK, CHALF), jnp.float32),
        pltpu.VMEM((BLK, CHALF), jnp.float32),
        pltpu.VMEM((BLK, CW), jnp.float32),
        pltpu.VMEM((SEG_PT, CHALF), jnp.float32),
        pltpu.VMEM((SEG_PT, CW), jnp.float32),
        pltpu.VMEM((SEG_PT, CHALF), jnp.float32),
        pltpu.VMEM((SEG_PT, CHALF), jnp.float32),
        pltpu.VMEM((SEG_PT, CW), jnp.float32),
        pltpu.VMEM((SEG_PT, CW), jnp.float32),
        pltpu.VMEM((SEG_PT, CHALF), jnp.float32),
        pltpu.VMEM_SHARED((NSEG, CHALF), jnp.float32),
        pltpu.VMEM_SHARED((NSEG, CHALF), jnp.float32),
        pltpu.VMEM_SHARED((NSEG, CW), jnp.float32),
        pltpu.VMEM_SHARED((NSEG, CW), jnp.float32),
        pltpu.SemaphoreType.DMA,
        pltpu.SemaphoreType.DMA,
        pltpu.SemaphoreType.DMA,
        pltpu.SemaphoreType.DMA,
        pltpu.SemaphoreType.DMA,
        pltpu.SemaphoreType.DMA,
        pltpu.SemaphoreType.DMA,
        pltpu.SemaphoreType.DMA,
    ],
)(_sc_body)


def _tc_body(ids_ref, x_ref, sums_ref, cnt_ref):
    i = pl.program_id(0)

    @pl.when(i == 0)
    def _init():
        sums_ref[...] = jnp.zeros_like(sums_ref)
        cnt_ref[...] = jnp.zeros_like(cnt_ref)

    seg = lax.broadcasted_iota(jnp.int32, (NSEG, BT), 0)
    oh = (seg == ids_ref[0]).astype(jnp.float32)
    sums_ref[...] += jax.lax.dot(oh, x_ref[...],
                                 precision=jax.lax.Precision.HIGHEST,
                                 preferred_element_type=jnp.float32)
    cnt_ref[...] += jnp.sum(oh, axis=1)[None, :]


_tc_pool = pl.pallas_call(
    _tc_body,
    grid=(NBT,),
    in_specs=[
        pl.BlockSpec((1, 1, BT), lambda i: (i, 0, 0)),
        pl.BlockSpec((BT, NCOL), lambda i: (R_SC // BT + i, 0)),
    ],
    out_specs=[
        pl.BlockSpec((NSEG, NCOL), lambda i: (0, 0)),
        pl.BlockSpec((1, NSEG), lambda i: (0, 0)),
    ],
    out_shape=[
        jax.ShapeDtypeStruct((NSEG, NCOL), jnp.float32),
        jax.ShapeDtypeStruct((1, NSEG), jnp.float32),
    ],
)


def _combine_body(ss_ref, sc_ref, ts_ref, tc_ref, o_ref):
    cnt = sc_ref[0, :, 0:1] + tc_ref[0][:, None]
    o_ref[...] = (ss_ref[...] + ts_ref[...]) / jnp.maximum(cnt, 1.0)


_combine = pl.pallas_call(
    _combine_body,
    out_shape=jax.ShapeDtypeStruct((NSEG, NCOL), jnp.float32),
)


@jax.jit
def kernel(x, batch):
    ids = batch.astype(jnp.int32)
    ids_sc = ids[:R_SC].reshape(NS, NB, BLK)
    ids_tc = ids[R_SC:].reshape(NBT, 1, BT)
    sc_sums, sc_counts = _sc_pool(x, ids_sc)
    tc_sums, tc_counts = _tc_pool(ids_tc, x)
    return _combine(sc_sums, sc_counts, tc_sums, tc_counts)


# speedup vs baseline: 1.7031x; 1.7031x over previous
import functools

import jax
import jax.numpy as jnp
from jax import lax
from jax.experimental import pallas as pl
from jax.experimental.pallas import tpu as pltpu
from jax.experimental.pallas import tpu_sc as plsc

NSEG = 512
NROW = 320000
NCOL = 128
NC = 2
NS = 16
NW = NC * NS
RPT = NROW // NW
BLK = 125
NB = RPT // BLK
SEG_PT = NSEG // NS
CW = 16


def _sc_body(x_hbm, ids_hbm, sums_hbm, counts_hbm,
             ids_v, xb0, xb1, xb2, xb3, ones_v, zb, zc, sbuf, tbuf, cbuf,
             ctbuf, obuf, tab_e, tab_o, cnt_e, cnt_o,
             sl0, sl1, sl2, sl3, ste, sto, sce, sco):
    c = lax.axis_index("c")
    s = lax.axis_index("s")
    wid = c * NS + s
    row0 = wid * RPT
    seg0 = s * SEG_PT

    xbs = (xb0, xb1, xb2, xb3)
    sls = (sl0, sl1, sl2, sl3)
    tabs = (tab_e, tab_o)
    cnts = (cnt_e, cnt_o)
    sts = (ste, sto)
    scs = (sce, sco)

    pltpu.sync_copy(ids_hbm.at[wid], ids_v)

    one16 = jnp.ones((16,), jnp.float32)
    zero16 = jnp.zeros((16,), jnp.float32)
    for r in range(BLK):
        ones_v[r, :] = one16
    for r in range(SEG_PT):
        zc[r, :] = zero16
    for r in range(SEG_PT):
        for j in range(NCOL // 16):
            zb[r, pl.ds(j * 16, 16)] = zero16

    pltpu.sync_copy(zb, tab_e.at[pl.ds(seg0, SEG_PT), :])
    pltpu.sync_copy(zb, tab_o.at[pl.ds(seg0, SEG_PT), :])
    pltpu.sync_copy(zc, cnt_e.at[pl.ds(seg0, SEG_PT), :])
    pltpu.sync_copy(zc, cnt_o.at[pl.ds(seg0, SEG_PT), :])
    plsc.subcore_barrier()

    def start_load(b_idx, k):
        pltpu.async_copy(
            x_hbm.at[pl.ds(row0 + b_idx * BLK, BLK), :], xbs[k], sls[k])

    def wait_load(k):
        pltpu.make_async_copy(
            x_hbm.at[pl.ds(0, BLK), :], xbs[k], sls[k]).wait()

    def start_scatter(b_idx, k):
        p = k % 2
        idx = ids_v.at[b_idx]
        pltpu.async_copy(ones_v, cnts[p].at[idx], scs[p], add=True)
        pltpu.async_copy(xbs[k], tabs[p].at[idx], sts[p], add=True)

    def wait_scatter(p):
        pltpu.make_async_copy(ones_v, cnts[p].at[ids_v.at[0]], scs[p]).wait()
        pltpu.make_async_copy(xbs[p], tabs[p].at[ids_v.at[0]], sts[p]).wait()

    start_load(0, 0)
    start_load(1, 1)
    start_load(2, 2)
    wait_load(0)
    start_scatter(0, 0)
    start_load(3, 3)
    wait_load(1)
    start_scatter(1, 1)
    wait_scatter(0)
    start_load(4, 0)
    wait_load(2)
    start_scatter(2, 2)
    wait_scatter(1)
    start_load(5, 1)
    wait_load(3)
    start_scatter(3, 3)

    def quad(i, carry):
        g = i * 4
        for k in range(4):
            wait_scatter(k % 2)
            start_load(g + k + 2, (k + 2) % 4)
            wait_load(k)
            start_scatter(g + k, k)
        return carry

    lax.fori_loop(1, NB // 4 - 1, quad, 0)

    for k in range(4):
        g = NB - 4 + k
        wait_scatter(k % 2)
        if k < 2:
            start_load(NB - 2 + k, k + 2)
        wait_load(k)
        start_scatter(g, k)
    wait_scatter(0)
    wait_scatter(1)

    plsc.subcore_barrier()

    pltpu.sync_copy(tab_e.at[pl.ds(seg0, SEG_PT), :], sbuf)
    pltpu.sync_copy(tab_o.at[pl.ds(seg0, SEG_PT), :], tbuf)
    pltpu.sync_copy(cnt_e.at[pl.ds(seg0, SEG_PT), :], cbuf)
    pltpu.sync_copy(cnt_o.at[pl.ds(seg0, SEG_PT), :], ctbuf)
    for r in range(SEG_PT):
        cbuf[r, :] += ctbuf[r, :]
        for j in range(NCOL // 16):
            sl = pl.ds(j * 16, 16)
            obuf[r, sl] = sbuf[r, sl] + tbuf[r, sl]
    pltpu.sync_copy(obuf, sums_hbm.at[c, pl.ds(seg0, SEG_PT), :])
    pltpu.sync_copy(cbuf, counts_hbm.at[c, pl.ds(seg0, SEG_PT), :])


_sc_pool = functools.partial(
    pl.kernel,
    out_type=(jax.ShapeDtypeStruct((NC, NSEG, NCOL), jnp.float32),
              jax.ShapeDtypeStruct((NC, NSEG, CW), jnp.float32)),
    mesh=plsc.VectorSubcoreMesh(core_axis_name="c", subcore_axis_name="s",
                                num_cores=NC, num_subcores=NS),
    compiler_params=pltpu.CompilerParams(use_tc_tiling_on_sc=False,
                                         needs_layout_passes=False),
    scratch_types=[
        pltpu.VMEM((NB, BLK), jnp.int32),
        pltpu.VMEM((BLK, NCOL), jnp.float32),
        pltpu.VMEM((BLK, NCOL), jnp.float32),
        pltpu.VMEM((BLK, NCOL), jnp.float32),
        pltpu.VMEM((BLK, NCOL), jnp.float32),
        pltpu.VMEM((BLK, CW), jnp.float32),
        pltpu.VMEM((SEG_PT, NCOL), jnp.float32),
        pltpu.VMEM((SEG_PT, CW), jnp.float32),
        pltpu.VMEM((SEG_PT, NCOL), jnp.float32),
        pltpu.VMEM((SEG_PT, NCOL), jnp.float32),
        pltpu.VMEM((SEG_PT, CW), jnp.float32),
        pltpu.VMEM((SEG_PT, CW), jnp.float32),
        pltpu.VMEM((SEG_PT, NCOL), jnp.float32),
        pltpu.VMEM_SHARED((NSEG, NCOL), jnp.float32),
        pltpu.VMEM_SHARED((NSEG, NCOL), jnp.float32),
        pltpu.VMEM_SHARED((NSEG, CW), jnp.float32),
        pltpu.VMEM_SHARED((NSEG, CW), jnp.float32),
        pltpu.SemaphoreType.DMA,
        pltpu.SemaphoreType.DMA,
        pltpu.SemaphoreType.DMA,
        pltpu.SemaphoreType.DMA,
        pltpu.SemaphoreType.DMA,
        pltpu.SemaphoreType.DMA,
        pltpu.SemaphoreType.DMA,
        pltpu.SemaphoreType.DMA,
    ],
)(_sc_body)


def _combine_body(s_ref, c_ref, o_ref):
    cnt = c_ref[0, :, 0:1] + c_ref[1, :, 0:1]
    o_ref[...] = (s_ref[0] + s_ref[1]) / jnp.maximum(cnt, 1.0)


_combine = pl.pallas_call(
    _combine_body,
    out_shape=jax.ShapeDtypeStruct((NSEG, NCOL), jnp.float32),
)


@jax.jit
def kernel(x, batch):
    ids = batch.astype(jnp.int32).reshape(NW, NB, BLK)
    sums, counts = _sc_pool(x, ids)
    return _combine(sums, counts)
